# Initial kernel scaffold; baseline (speedup 1.0000x reference)
import jax
import jax.numpy as jnp
from jax.experimental import pallas as pl


def kernel(x, node_features, node_batch, token_index, bn1_w, bn1_b, w1, b1, w2, b2, bn2_w, bn2_b, w3, b3, w4, b4):
    raise NotImplementedError("write your pallas kernel here")


# speedup vs baseline: 3.9684x; 3.9684x over previous
import functools

import jax
import jax.numpy as jnp
from jax import lax
from jax.experimental import pallas as pl
from jax.experimental.pallas import tpu as pltpu
from jax.experimental.pallas import tpu_sc as plsc

D = 128
NTOK = 320000
NNODES = 10000
NPAD = 10240
TOT = 2 * NTOK

_STATS_R = 4000
_MLP_R = 2000
_SC_CHUNK = 80
_NS = 16


def _stats_body(x_ref, s_ref, q_ref):
    i = pl.program_id(0)
    xb = x_ref[...]

    @pl.when(i == 0)
    def _():
        s_ref[...] = jnp.zeros_like(s_ref)
        q_ref[...] = jnp.zeros_like(q_ref)

    s_ref[...] += jnp.sum(xb, axis=0, keepdims=True)
    q_ref[...] += jnp.sum(xb * xb, axis=0, keepdims=True)


def _stats(x):
    return pl.pallas_call(
        _stats_body,
        grid=(NTOK // _STATS_R,),
        in_specs=[pl.BlockSpec((_STATS_R, D), lambda i: (i, 0))],
        out_specs=(pl.BlockSpec((1, D), lambda i: (0, 0)),
                   pl.BlockSpec((1, D), lambda i: (0, 0))),
        out_shape=(jax.ShapeDtypeStruct((1, D), jnp.float32),
                   jax.ShapeDtypeStruct((1, D), jnp.float32)),
    )(x)


def _mlp_body(s_ref, q_ref, bnw_ref, bnb_ref, w1_ref, b1_ref, w2_ref, b2_ref,
              x_ref, h_ref):
    mu = s_ref[...] / NTOK
    var = q_ref[...] / NTOK - mu * mu
    scale = bnw_ref[...] * lax.rsqrt(var + 1e-5)
    shift = bnb_ref[...] - mu * scale
    xn = (x_ref[...] * scale + shift).astype(jnp.bfloat16)
    g = lax.dot_general(xn, w1_ref[...], (((1,), (1,)), ((), ())),
                        preferred_element_type=jnp.float32)
    g = jnp.maximum(g + b1_ref[...], 0.0).astype(jnp.bfloat16)
    h = lax.dot_general(g, w2_ref[...], (((1,), (1,)), ((), ())),
                        preferred_element_type=jnp.float32)
    h = h + b2_ref[...]
    h_ref[0, :, :] = h[:, :D]
    h_ref[1, :, :] = h[:, D:]


def _mlp(x, s, q, bn1_w, bn1_b, w1, b1, w2, b2):
    full = lambda shape: pl.BlockSpec(shape, lambda i: tuple(0 for _ in shape))
    return pl.pallas_call(
        _mlp_body,
        grid=(NTOK // _MLP_R,),
        in_specs=[full((1, D)), full((1, D)), full((1, D)), full((1, D)),
                  full((2 * D, D)), full((1, 2 * D)),
                  full((2 * D, 2 * D)), full((1, 2 * D)),
                  pl.BlockSpec((_MLP_R, D), lambda i: (i, 0))],
        out_specs=pl.BlockSpec((2, _MLP_R, D), lambda i: (0, i, 0)),
        out_shape=jax.ShapeDtypeStruct((2, NTOK, D), jnp.float32),
    )(s, q, bn1_w.reshape(1, D), bn1_b.reshape(1, D),
      w1.astype(jnp.bfloat16), b1.reshape(1, 2 * D),
      w2.astype(jnp.bfloat16), b2.reshape(1, 2 * D), x)


def _segment_sum(h_rows, idx, zeros):
    mesh = plsc.VectorSubcoreMesh(core_axis_name="c", subcore_axis_name="s")

    @functools.partial(
        pl.kernel,
        mesh=mesh,
        out_type=jax.ShapeDtypeStruct((2, NPAD, D), jnp.float32),
        scratch_types=[pltpu.VMEM_SHARED((NPAD, D), jnp.float32)],
    )
    def scatter_kernel(h_hbm, idx_hbm, z_hbm, out_hbm, acc):
        core = lax.axis_index("c")
        sid = lax.axis_index("s")
        rows_per_tile = NPAD // _NS
        sl = pl.ds(sid * rows_per_tile, rows_per_tile)
        pltpu.sync_copy(z_hbm.at[sl], acc.at[sl])
        plsc.subcore_barrier()

        def body(rows_v, idx_v):
            pltpu.sync_copy(rows_v, acc.at[idx_v.at[0, 0]], add=True)

        pltpu.emit_pipeline(
            body,
            grid=(TOT // _SC_CHUNK,),
            in_specs=[pl.BlockSpec((_SC_CHUNK, D), lambda i: (i, 0)),
                      pl.BlockSpec((1, 1, _SC_CHUNK), lambda i: (i, 0, 0))],
            core_axis_name=("c", "s"),
            dimension_semantics=(pltpu.PARALLEL,),
        )(h_hbm, idx_hbm)
        plsc.subcore_barrier()
        pltpu.sync_copy(acc.at[sl], out_hbm.at[core].at[sl])

    return scatter_kernel(h_rows, idx, zeros)


def _node_body(p_ref, bnw_ref, bnb_ref, w3_ref, b3_ref, w4_ref, b4_ref, o_ref):
    nf = p_ref[0, :NNODES, :] + p_ref[1, :NNODES, :]
    mu = jnp.mean(nf, axis=0, keepdims=True)
    var = jnp.mean(nf * nf, axis=0, keepdims=True) - mu * mu
    xn = ((nf - mu) * lax.rsqrt(var + 1e-5) * bnw_ref[...]
          + bnb_ref[...]).astype(jnp.bfloat16)
    g = lax.dot_general(xn, w3_ref[...], (((1,), (1,)), ((), ())),
                        preferred_element_type=jnp.float32)
    g = jnp.maximum(g + b3_ref[...], 0.0).astype(jnp.bfloat16)
    o = lax.dot_general(g, w4_ref[...], (((1,), (1,)), ((), ())),
                        preferred_element_type=jnp.float32)
    o_ref[...] = o + b4_ref[...]


def _node_mlp(partials, bn2_w, bn2_b, w3, b3, w4, b4):
    return pl.pallas_call(
        _node_body,
        out_shape=jax.ShapeDtypeStruct((NNODES, D), jnp.float32),
    )(partials, bn2_w.reshape(1, D), bn2_b.reshape(1, D),
      w3.astype(jnp.bfloat16), b3.reshape(1, D),
      w4.astype(jnp.bfloat16), b4.reshape(1, D))


def kernel(x, node_features, node_batch, token_index, bn1_w, bn1_b, w1, b1,
           w2, b2, bn2_w, bn2_b, w3, b3, w4, b4):
    s, q = _stats(x)
    h = _mlp(x, s, q, bn1_w, bn1_b, w1, b1, w2, b2)
    h_rows = h.reshape(TOT, D)
    idx = token_index.reshape(TOT // _SC_CHUNK, 1, _SC_CHUNK)
    zeros = jnp.zeros((NPAD, D), jnp.float32)
    partials = _segment_sum(h_rows, idx, zeros)
    return _node_mlp(partials, bn2_w, bn2_b, w3, b3, w4, b4)


# speedup vs baseline: 4.4049x; 1.1100x over previous
import functools

import jax
import jax.numpy as jnp
from jax import lax
from jax.experimental import pallas as pl
from jax.experimental.pallas import tpu as pltpu
from jax.experimental.pallas import tpu_sc as plsc

D = 128
NTOK = 320000
NNODES = 10000
NPAD = 10240
TOT = 2 * NTOK

_STATS_R = 4000
_MLP_R = 2000
_SC_CHUNK = 80
_NS = 16
_P = 4
_CHTOK = NTOK // _P


def _stats_body(x_ref, s_ref, q_ref):
    i = pl.program_id(0)
    xb = x_ref[...]

    @pl.when(i == 0)
    def _():
        s_ref[...] = jnp.zeros_like(s_ref)
        q_ref[...] = jnp.zeros_like(q_ref)

    s_ref[...] += jnp.sum(xb, axis=0, keepdims=True)
    q_ref[...] += jnp.sum(xb * xb, axis=0, keepdims=True)


def _stats(x):
    return pl.pallas_call(
        _stats_body,
        grid=(NTOK // _STATS_R,),
        in_specs=[pl.BlockSpec((_STATS_R, D), lambda i: (i, 0))],
        out_specs=(pl.BlockSpec((1, D), lambda i: (0, 0)),
                   pl.BlockSpec((1, D), lambda i: (0, 0))),
        out_shape=(jax.ShapeDtypeStruct((1, D), jnp.float32),
                   jax.ShapeDtypeStruct((1, D), jnp.float32)),
    )(x)


def _mlp_body(s_ref, q_ref, bnw_ref, bnb_ref, w1_ref, b1_ref, w2_ref, b2_ref,
              x_ref, h_ref):
    mu = s_ref[...] / NTOK
    var = q_ref[...] / NTOK - mu * mu
    scale = bnw_ref[...] * lax.rsqrt(var + 1e-5)
    shift = bnb_ref[...] - mu * scale
    xn = (x_ref[...] * scale + shift).astype(jnp.bfloat16)
    g = lax.dot_general(xn, w1_ref[...], (((1,), (1,)), ((), ())),
                        preferred_element_type=jnp.float32)
    g = jnp.maximum(g + b1_ref[...], 0.0).astype(jnp.bfloat16)
    h = lax.dot_general(g, w2_ref[...], (((1,), (1,)), ((), ())),
                        preferred_element_type=jnp.float32)
    h = h + b2_ref[...]
    h_ref[0, :, :] = h[:, :D]
    h_ref[1, :, :] = h[:, D:]


def _mlp(x, p, s, q, bn1_w, bn1_b, w1, b1, w2, b2):
    full = lambda shape: pl.BlockSpec(shape, lambda i: tuple(0 for _ in shape))
    base = p * (_CHTOK // _MLP_R)
    return pl.pallas_call(
        _mlp_body,
        grid=(_CHTOK // _MLP_R,),
        in_specs=[full((1, D)), full((1, D)), full((1, D)), full((1, D)),
                  full((2 * D, D)), full((1, 2 * D)),
                  full((2 * D, 2 * D)), full((1, 2 * D)),
                  pl.BlockSpec((_MLP_R, D), lambda i: (base + i, 0))],
        out_specs=pl.BlockSpec((2, _MLP_R, D), lambda i: (0, i, 0)),
        out_shape=jax.ShapeDtypeStruct((2, _CHTOK, D), jnp.float32),
    )(s, q, bn1_w.reshape(1, D), bn1_b.reshape(1, D),
      w1.astype(jnp.bfloat16), b1.reshape(1, 2 * D),
      w2.astype(jnp.bfloat16), b2.reshape(1, 2 * D), x)


def _segment_sum(h_rows, idx, zeros):
    mesh = plsc.VectorSubcoreMesh(core_axis_name="c", subcore_axis_name="s")

    @functools.partial(
        pl.kernel,
        mesh=mesh,
        out_type=jax.ShapeDtypeStruct((2, NPAD, D), jnp.float32),
        scratch_types=[pltpu.VMEM_SHARED((NPAD, D), jnp.float32)],
    )
    def scatter_kernel(h_hbm, idx_hbm, z_hbm, out_hbm, acc):
        core = lax.axis_index("c")
        sid = lax.axis_index("s")
        rows_per_tile = NPAD // _NS
        sl = pl.ds(sid * rows_per_tile, rows_per_tile)
        pltpu.sync_copy(z_hbm.at[sl], acc.at[sl])
        plsc.subcore_barrier()

        def body(rows_v, idx_v):
            pltpu.sync_copy(rows_v, acc.at[idx_v.at[0, 0]], add=True)

        pltpu.emit_pipeline(
            body,
            grid=(2 * _CHTOK // _SC_CHUNK,),
            in_specs=[pl.BlockSpec((_SC_CHUNK, D), lambda i: (i, 0)),
                      pl.BlockSpec((1, 1, _SC_CHUNK), lambda i: (i, 0, 0))],
            core_axis_name=("c", "s"),
            dimension_semantics=(pltpu.PARALLEL,),
        )(h_hbm, idx_hbm)
        plsc.subcore_barrier()
        pltpu.sync_copy(acc.at[sl], out_hbm.at[core].at[sl])

    return scatter_kernel(h_rows, idx, zeros)


def _node_body(*refs):
    p_refs = refs[:_P]
    bnw_ref, bnb_ref, w3_ref, b3_ref, w4_ref, b4_ref, o_ref = refs[_P:]
    nf = p_refs[0][0, :NNODES, :] + p_refs[0][1, :NNODES, :]
    for p_ref in p_refs[1:]:
        nf = nf + p_ref[0, :NNODES, :] + p_ref[1, :NNODES, :]
    mu = jnp.mean(nf, axis=0, keepdims=True)
    var = jnp.mean(nf * nf, axis=0, keepdims=True) - mu * mu
    xn = ((nf - mu) * lax.rsqrt(var + 1e-5) * bnw_ref[...]
          + bnb_ref[...]).astype(jnp.bfloat16)
    g = lax.dot_general(xn, w3_ref[...], (((1,), (1,)), ((), ())),
                        preferred_element_type=jnp.float32)
    g = jnp.maximum(g + b3_ref[...], 0.0).astype(jnp.bfloat16)
    o = lax.dot_general(g, w4_ref[...], (((1,), (1,)), ((), ())),
                        preferred_element_type=jnp.float32)
    o_ref[...] = o + b4_ref[...]


def _node_mlp(partials, bn2_w, bn2_b, w3, b3, w4, b4):
    return pl.pallas_call(
        _node_body,
        out_shape=jax.ShapeDtypeStruct((NNODES, D), jnp.float32),
    )(*partials, bn2_w.reshape(1, D), bn2_b.reshape(1, D),
      w3.astype(jnp.bfloat16), b3.reshape(1, D),
      w4.astype(jnp.bfloat16), b4.reshape(1, D))


def kernel(x, node_features, node_batch, token_index, bn1_w, bn1_b, w1, b1,
           w2, b2, bn2_w, bn2_b, w3, b3, w4, b4):
    s, q = _stats(x)
    zeros = jnp.zeros((NPAD, D), jnp.float32)
    partials = []
    for p in range(_P):
        h = _mlp(x, p, s, q, bn1_w, bn1_b, w1, b1, w2, b2)
        h_rows = h.reshape(2 * _CHTOK, D)
        idx = token_index[:, p * _CHTOK:(p + 1) * _CHTOK].reshape(
            2 * _CHTOK // _SC_CHUNK, 1, _SC_CHUNK)
        partials.append(_segment_sum(h_rows, idx, zeros))
    return _node_mlp(partials, bn2_w, bn2_b, w3, b3, w4, b4)


# speedup vs baseline: 4.7870x; 1.0867x over previous
import functools

import jax
import jax.numpy as jnp
from jax import lax
from jax.experimental import pallas as pl
from jax.experimental.pallas import tpu as pltpu
from jax.experimental.pallas import tpu_sc as plsc

D = 128
NTOK = 320000
NNODES = 10000
NPAD = 10240
TOT = 2 * NTOK

_STATS_R = 8000
_MLP_R = 4000
_SC_CHUNK = 128
_NS = 16
_P = 4
_CHTOK = NTOK // _P


def _stats_body(x_ref, s_ref, q_ref):
    i = pl.program_id(0)
    xb = x_ref[...]

    @pl.when(i == 0)
    def _():
        s_ref[...] = jnp.zeros_like(s_ref)
        q_ref[...] = jnp.zeros_like(q_ref)

    s_ref[...] += jnp.sum(xb, axis=0, keepdims=True)
    q_ref[...] += jnp.sum(xb * xb, axis=0, keepdims=True)


def _stats(x):
    return pl.pallas_call(
        _stats_body,
        grid=(NTOK // _STATS_R,),
        in_specs=[pl.BlockSpec((_STATS_R, D), lambda i: (i, 0))],
        out_specs=(pl.BlockSpec((1, D), lambda i: (0, 0)),
                   pl.BlockSpec((1, D), lambda i: (0, 0))),
        out_shape=(jax.ShapeDtypeStruct((1, D), jnp.float32),
                   jax.ShapeDtypeStruct((1, D), jnp.float32)),
    )(x)


def _mlp_body(s_ref, q_ref, bnw_ref, bnb_ref, w1_ref, b1_ref, w2_ref, b2_ref,
              x_ref, h_ref):
    mu = s_ref[...] / NTOK
    var = q_ref[...] / NTOK - mu * mu
    scale = bnw_ref[...] * lax.rsqrt(var + 1e-5)
    shift = bnb_ref[...] - mu * scale
    xn = (x_ref[...] * scale + shift).astype(jnp.bfloat16)
    g = lax.dot_general(xn, w1_ref[...], (((1,), (1,)), ((), ())),
                        preferred_element_type=jnp.float32)
    g = jnp.maximum(g + b1_ref[...], 0.0).astype(jnp.bfloat16)
    h = lax.dot_general(g, w2_ref[...], (((1,), (1,)), ((), ())),
                        preferred_element_type=jnp.float32)
    h = h + b2_ref[...]
    h_ref[0, :, :] = h[:, :D]
    h_ref[1, :, :] = h[:, D:]


def _mlp(x, p, s, q, bn1_w, bn1_b, w1, b1, w2, b2):
    full = lambda shape: pl.BlockSpec(shape, lambda i: tuple(0 for _ in shape))
    base = p * (_CHTOK // _MLP_R)
    return pl.pallas_call(
        _mlp_body,
        grid=(_CHTOK // _MLP_R,),
        in_specs=[full((1, D)), full((1, D)), full((1, D)), full((1, D)),
                  full((2 * D, D)), full((1, 2 * D)),
                  full((2 * D, 2 * D)), full((1, 2 * D)),
                  pl.BlockSpec((_MLP_R, D), lambda i: (base + i, 0))],
        out_specs=pl.BlockSpec((2, _MLP_R, D), lambda i: (0, i, 0)),
        out_shape=jax.ShapeDtypeStruct((2, _CHTOK, D), jnp.float32),
    )(s, q, bn1_w.reshape(1, D), bn1_b.reshape(1, D),
      w1.astype(jnp.bfloat16), b1.reshape(1, 2 * D),
      w2.astype(jnp.bfloat16), b2.reshape(1, 2 * D), x)


def _segment_sum(h_rows, idx, zeros):
    mesh = plsc.VectorSubcoreMesh(core_axis_name="c", subcore_axis_name="s")

    @functools.partial(
        pl.kernel,
        mesh=mesh,
        out_type=jax.ShapeDtypeStruct((2, NPAD, D), jnp.float32),
        scratch_types=[pltpu.VMEM_SHARED((NPAD, D), jnp.float32)],
    )
    def scatter_kernel(h_hbm, idx_hbm, z_hbm, out_hbm, acc):
        core = lax.axis_index("c")
        sid = lax.axis_index("s")
        rows_per_tile = NPAD // _NS
        sl = pl.ds(sid * rows_per_tile, rows_per_tile)
        pltpu.sync_copy(z_hbm.at[sl], acc.at[sl])
        plsc.subcore_barrier()

        def body(rows_v, idx_v):
            pltpu.sync_copy(rows_v, acc.at[idx_v.at[0, 0]], add=True)

        pltpu.emit_pipeline(
            body,
            grid=(2 * _CHTOK // _SC_CHUNK,),
            in_specs=[pl.BlockSpec((_SC_CHUNK, D), lambda i: (i, 0)),
                      pl.BlockSpec((1, 1, _SC_CHUNK), lambda i: (i, 0, 0))],
            core_axis_name=("c", "s"),
            dimension_semantics=(pltpu.PARALLEL,),
        )(h_hbm, idx_hbm)
        plsc.subcore_barrier()
        pltpu.sync_copy(acc.at[sl], out_hbm.at[core].at[sl])

    return scatter_kernel(h_rows, idx, zeros)


def _node_body(*refs):
    p_refs = refs[:_P]
    bnw_ref, bnb_ref, w3_ref, b3_ref, w4_ref, b4_ref, o_ref = refs[_P:]
    nf = p_refs[0][0, :NNODES, :] + p_refs[0][1, :NNODES, :]
    for p_ref in p_refs[1:]:
        nf = nf + p_ref[0, :NNODES, :] + p_ref[1, :NNODES, :]
    mu = jnp.mean(nf, axis=0, keepdims=True)
    var = jnp.mean(nf * nf, axis=0, keepdims=True) - mu * mu
    xn = ((nf - mu) * lax.rsqrt(var + 1e-5) * bnw_ref[...]
          + bnb_ref[...]).astype(jnp.bfloat16)
    g = lax.dot_general(xn, w3_ref[...], (((1,), (1,)), ((), ())),
                        preferred_element_type=jnp.float32)
    g = jnp.maximum(g + b3_ref[...], 0.0).astype(jnp.bfloat16)
    o = lax.dot_general(g, w4_ref[...], (((1,), (1,)), ((), ())),
                        preferred_element_type=jnp.float32)
    o_ref[...] = o + b4_ref[...]


def _node_mlp(partials, bn2_w, bn2_b, w3, b3, w4, b4):
    return pl.pallas_call(
        _node_body,
        out_shape=jax.ShapeDtypeStruct((NNODES, D), jnp.float32),
    )(*partials, bn2_w.reshape(1, D), bn2_b.reshape(1, D),
      w3.astype(jnp.bfloat16), b3.reshape(1, D),
      w4.astype(jnp.bfloat16), b4.reshape(1, D))


def kernel(x, node_features, node_batch, token_index, bn1_w, bn1_b, w1, b1,
           w2, b2, bn2_w, bn2_b, w3, b3, w4, b4):
    s, q = _stats(x)
    zeros = jnp.zeros((NPAD, D), jnp.float32)
    partials = []
    for p in range(_P):
        h = _mlp(x, p, s, q, bn1_w, bn1_b, w1, b1, w2, b2)
        h_rows = h.reshape(2 * _CHTOK, D)
        idx = token_index[:, p * _CHTOK:(p + 1) * _CHTOK].reshape(
            2 * _CHTOK // _SC_CHUNK, 1, _SC_CHUNK)
        partials.append(_segment_sum(h_rows, idx, zeros))
    return _node_mlp(partials, bn2_w, bn2_b, w3, b3, w4, b4)
